# baseline (speedup 1.0000x reference)
import functools

import jax
import jax.numpy as jnp
from jax import lax
from jax.experimental import pallas as pl
from jax.experimental.pallas import tpu as pltpu
from jax.experimental.pallas import tpu_sc as plsc

D = 64
NT = 200
NB = 4096
B = NB * NT
BLK = 128


NCOL = 1000000 // 128 + 1


@functools.lru_cache(maxsize=None)
def _build_relayout():
    info = plsc.get_sparse_core_info()
    nc, ns = info.num_cores, info.num_subcores
    nw = nc * ns
    n_main = (NCOL - 5) // (2 * nw)

    mesh = plsc.VectorSubcoreMesh(core_axis_name="c", subcore_axis_name="s")

    @functools.partial(
        pl.kernel,
        out_type=jax.ShapeDtypeStruct((500000, 128), jnp.float32),
        mesh=mesh,
        compiler_params=pltpu.CompilerParams(
            use_tc_tiling_on_sc=True, needs_layout_passes=False,
            disable_bounds_checks=True),
        scratch_types=[
            pltpu.VMEM((2, D, 256), jnp.float32),
            pltpu.VMEM((2, 128, 128), jnp.float32),
            pltpu.SemaphoreType.DMA,
            pltpu.SemaphoreType.DMA,
        ],
    )
    def relayout_kernel(tt_hbm, tp_hbm, in_v, out_v, isem, osem):
        wid = lax.axis_index("s") * nc + lax.axis_index("c")
        iota = lax.iota(jnp.int32, 16)
        hvec = lax.shift_right_logical(iota, 3)
        h64 = hvec * 64
        p8 = iota & 7

        def fire_in(ch, buf):
            pltpu.async_copy(
                tt_hbm.at[:, pl.ds(ch * 256, 256)], in_v.at[buf], isem)

        def drain_in(buf):
            pltpu.make_async_copy(
                tt_hbm.at[:, pl.ds(0, 256)], in_v.at[buf], isem).wait()

        def drain_out():
            pltpu.make_async_copy(
                out_v.at[0], tp_hbm.at[pl.ds(0, 128)], osem).wait()

        def shuffle(buf, npg):
            def body(d0, carry):
                for u in range(4):
                    dvec = (d0 * 4 + u + iota) & 63
                    dst_lane = h64 + dvec
                    for pg in range(npg):
                        pvec = p8 + pg * 8
                        src_lane = 2 * pvec + hvec
                        vals = plsc.load_gather(in_v.at[buf], [dvec, src_lane])
                        plsc.store_scatter(out_v.at[buf], [pvec, dst_lane], vals)
                return carry
            lax.fori_loop(0, D // 4, body, 0)

        fire_in(wid, 0)

        def pair_body(q, carry):
            for sub in range(2):
                ci = 2 * q + sub
                buf = sub
                ch = wid + ci * nw

                @pl.when(ci < n_main - 1)
                def _():
                    fire_in(ch + nw, 1 - buf)

                drain_in(buf)

                @pl.when(ci >= 2)
                def _():
                    drain_out()

                shuffle(buf, 16)
                pltpu.async_copy(
                    out_v.at[buf], tp_hbm.at[pl.ds(ch * 128, 128)], osem)
            return carry

        lax.fori_loop(0, n_main // 2, pair_body, 0)
        drain_out()
        drain_out()

        @pl.when(wid < 5)
        def _():
            c = (NCOL - 5) + wid
            pltpu.sync_copy(
                tt_hbm.at[:, pl.ds(c * 128, 128)],
                in_v.at[0].at[:, pl.ds(0, 128)])
            shuffle(0, 8)

            @pl.when(wid < 4)
            def _():
                pltpu.sync_copy(
                    out_v.at[0].at[pl.ds(0, D)],
                    tp_hbm.at[pl.ds(c * D, D)])

            @pl.when(wid == 4)
            def _():
                pltpu.sync_copy(
                    out_v.at[0].at[pl.ds(0, 32)],
                    tp_hbm.at[pl.ds(c * D, 32)])

    return relayout_kernel


@functools.lru_cache(maxsize=None)
def _build():
    info = plsc.get_sparse_core_info()
    nc, ns = info.num_cores, info.num_subcores
    nw = nc * ns
    b_per_w = B // nw
    n_blocks = b_per_w // BLK
    n_pairs = n_blocks // 2

    mesh = plsc.VectorSubcoreMesh(core_axis_name="c", subcore_axis_name="s")

    @functools.partial(
        pl.kernel,
        out_type=jax.ShapeDtypeStruct((NT, D, NB), jnp.float32),
        mesh=mesh,
        compiler_params=pltpu.CompilerParams(
            use_tc_tiling_on_sc=True, needs_layout_passes=False,
            disable_bounds_checks=True),
        scratch_types=[
            pltpu.VMEM((b_per_w,), jnp.int32),
            pltpu.VMEM((b_per_w,), jnp.int32),
            pltpu.VMEM((b_per_w,), jnp.int32),
            pltpu.VMEM((2, BLK, 128), jnp.float32),
            pltpu.VMEM((2, D, BLK), jnp.float32),
            pltpu.SemaphoreType.DMA,
            pltpu.SemaphoreType.DMA,
        ],
    )
    def gather_kernel(seq_hbm, table_hbm, out_hbm,
                      idx_v, idx2_v, hv_v, pair_v, outt_v, gsem, ssem):
        wid = lax.axis_index("s") * nc + lax.axis_index("c")
        j0 = wid * b_per_w
        pltpu.sync_copy(seq_hbm.at[pl.ds(j0, b_per_w)], idx_v)

        def prep(i, carry):
            x = idx_v[pl.ds(i * 16, 16)]
            idx2_v[pl.ds(i * 16, 16)] = lax.shift_right_logical(x, 1)
            hv_v[pl.ds(i * 16, 16)] = lax.shift_left(x & 1, 6)
            return carry

        lax.fori_loop(0, b_per_w // 16, prep, 0)

        def fire_gather(k, buf):
            pltpu.async_copy(
                table_hbm.at[idx2_v.at[pl.ds(k * BLK, BLK)]],
                pair_v.at[buf], gsem)

        def drain_gather(buf):
            pltpu.make_async_copy(
                table_hbm.at[idx2_v.at[pl.ds(0, BLK)]],
                pair_v.at[buf], gsem).wait()

        def drain_store():
            pltpu.make_async_copy(
                outt_v.at[0], out_hbm.at[0].at[:, pl.ds(0, BLK)], ssem).wait()

        fire_gather(0, 0)

        def pair_body(p, carry):
            for sub in range(2):
                k = 2 * p + sub
                buf = sub

                if sub == 0:
                    fire_gather(k + 1, 1 - buf)
                else:
                    @pl.when(p < n_pairs - 1)
                    def _():
                        fire_gather(k + 1, 1 - buf)

                drain_gather(buf)

                @pl.when(k >= 2)
                def _():
                    drain_store()

                blk_j = j0 + k * BLK
                t = blk_j // NB
                b0 = blk_j % NB
                koff = k * BLK
                iota = lax.iota(jnp.int32, 16)
                tok_g = [iota + g * 16 for g in range(BLK // 16)]
                hv_g = [hv_v[pl.ds(koff + g * 16, 16)]
                        for g in range(BLK // 16)]

                def shuffle_d(d0, carry2):
                    for u in range(4):
                        lane_d = (d0 * 4 + u + iota) & 63
                        for g in range(BLK // 16):
                            vals = plsc.load_gather(
                                pair_v.at[buf], [tok_g[g], hv_g[g] + lane_d])
                            plsc.store_scatter(
                                outt_v.at[buf], [lane_d, tok_g[g]], vals)
                    return carry2

                lax.fori_loop(0, D // 4, shuffle_d, 0)
                pltpu.async_copy(
                    outt_v.at[buf],
                    out_hbm.at[t].at[:, pl.ds(b0, BLK)], ssem)
            return carry

        lax.fori_loop(0, n_pairs, pair_body, 0)
        drain_store()
        drain_store()

    return gather_kernel


def kernel(seq, table):
    table2 = _build_relayout()(table.T)
    idx = seq.T.reshape(B)
    out = _build()(idx, table2)
    return out.transpose(2, 0, 1)


# speedup vs baseline: 1.8425x; 1.8425x over previous
import functools

import jax
import jax.numpy as jnp
from jax import lax
from jax.experimental import pallas as pl
from jax.experimental.pallas import tpu as pltpu
from jax.experimental.pallas import tpu_sc as plsc

D = 64
NT = 200
NB = 4096
B = NB * NT
BLK = 128


NCOL = 1000000 // 128 + 1


@functools.lru_cache(maxsize=None)
def _build_relayout():
    info = plsc.get_sparse_core_info()
    nc, ns = info.num_cores, info.num_subcores
    nw = nc * ns
    n_main = (NCOL - 5) // (2 * nw)

    mesh = plsc.VectorSubcoreMesh(core_axis_name="c", subcore_axis_name="s")

    @functools.partial(
        pl.kernel,
        out_type=jax.ShapeDtypeStruct((500000, 128), jnp.float32),
        mesh=mesh,
        compiler_params=pltpu.CompilerParams(
            use_tc_tiling_on_sc=True, needs_layout_passes=False,
            disable_bounds_checks=True),
        scratch_types=[
            pltpu.VMEM((2, D, 256), jnp.float32),
            pltpu.VMEM((2, 128, 128), jnp.float32),
            pltpu.VMEM((2, D * 256), jnp.float32),
            pltpu.VMEM((2, 128 * 128), jnp.float32),
            pltpu.SemaphoreType.DMA,
            pltpu.SemaphoreType.DMA,
        ],
    )
    def relayout_kernel(tt_hbm, tp_hbm, in_v, out_v, inf_v, outf_v,
                        isem, osem):
        wid = lax.axis_index("s") * nc + lax.axis_index("c")
        iota = lax.iota(jnp.int32, 16)
        hvec = lax.shift_right_logical(iota, 3)
        h64 = hvec * 64
        p8 = iota & 7

        def fire_in(ch, buf):
            pltpu.async_copy(
                tt_hbm.at[:, pl.ds(ch * 256, 256)], in_v.at[buf], isem)

        def drain_in(buf):
            pltpu.make_async_copy(
                tt_hbm.at[:, pl.ds(0, 256)], in_v.at[buf], isem).wait()

        def drain_out():
            pltpu.make_async_copy(
                out_v.at[0], tp_hbm.at[pl.ds(0, 128)], osem).wait()

        def shuffle(buf, npg):
            @plsc.parallel_loop(0, D // 4)
            def body(d0):
                for u in range(4):
                    dvec = (d0 * 4 + u + iota) & 63
                    dst_lane = h64 + dvec
                    for pg in range(npg):
                        pvec = p8 + pg * 8
                        src_lane = 2 * pvec + hvec
                        vals = plsc.load_gather(in_v.at[buf], [dvec, src_lane])
                        plsc.store_scatter(out_v.at[buf], [pvec, dst_lane], vals)

        fire_in(wid, 0)

        def pair_body(q, carry):
            for sub in range(2):
                ci = 2 * q + sub
                buf = sub
                ch = wid + ci * nw

                @pl.when(ci < n_main - 1)
                def _():
                    fire_in(ch + nw, 1 - buf)

                drain_in(buf)

                @pl.when(ci >= 2)
                def _():
                    drain_out()

                shuffle(buf, 16)
                pltpu.async_copy(
                    out_v.at[buf], tp_hbm.at[pl.ds(ch * 128, 128)], osem)
            return carry

        lax.fori_loop(0, n_main // 2, pair_body, 0)
        drain_out()
        drain_out()

        @pl.when(wid < 5)
        def _():
            c = (NCOL - 5) + wid
            pltpu.sync_copy(
                tt_hbm.at[:, pl.ds(c * 128, 128)],
                in_v.at[0].at[:, pl.ds(0, 128)])
            shuffle(0, 8)

            @pl.when(wid < 4)
            def _():
                pltpu.sync_copy(
                    out_v.at[0].at[pl.ds(0, D)],
                    tp_hbm.at[pl.ds(c * D, D)])

            @pl.when(wid == 4)
            def _():
                pltpu.sync_copy(
                    out_v.at[0].at[pl.ds(0, 32)],
                    tp_hbm.at[pl.ds(c * D, 32)])

    return relayout_kernel


@functools.lru_cache(maxsize=None)
def _build():
    info = plsc.get_sparse_core_info()
    nc, ns = info.num_cores, info.num_subcores
    nw = nc * ns
    b_per_w = B // nw
    n_blocks = b_per_w // BLK
    n_pairs = n_blocks // 2

    mesh = plsc.VectorSubcoreMesh(core_axis_name="c", subcore_axis_name="s")

    @functools.partial(
        pl.kernel,
        out_type=jax.ShapeDtypeStruct((NT, D, NB), jnp.float32),
        mesh=mesh,
        compiler_params=pltpu.CompilerParams(
            use_tc_tiling_on_sc=True, needs_layout_passes=False,
            disable_bounds_checks=True),
        scratch_types=[
            pltpu.VMEM((b_per_w,), jnp.int32),
            pltpu.VMEM((b_per_w,), jnp.int32),
            pltpu.VMEM((b_per_w,), jnp.int32),
            pltpu.VMEM((2, BLK, 128), jnp.float32),
            pltpu.VMEM((2, D, BLK), jnp.float32),
            pltpu.SemaphoreType.DMA,
            pltpu.SemaphoreType.DMA,
        ],
    )
    def gather_kernel(seq_hbm, table_hbm, out_hbm,
                      idx_v, idx2_v, hv_v, pair_v, outt_v, gsem, ssem):
        wid = lax.axis_index("s") * nc + lax.axis_index("c")
        j0 = wid * b_per_w
        pltpu.sync_copy(seq_hbm.at[pl.ds(j0, b_per_w)], idx_v)

        def prep(i, carry):
            x = idx_v[pl.ds(i * 16, 16)]
            idx2_v[pl.ds(i * 16, 16)] = lax.shift_right_logical(x, 1)
            hv_v[pl.ds(i * 16, 16)] = lax.shift_left(x & 1, 6)
            return carry

        lax.fori_loop(0, b_per_w // 16, prep, 0)

        def fire_gather(k, buf):
            pltpu.async_copy(
                table_hbm.at[idx2_v.at[pl.ds(k * BLK, BLK)]],
                pair_v.at[buf], gsem)

        def drain_gather(buf):
            pltpu.make_async_copy(
                table_hbm.at[idx2_v.at[pl.ds(0, BLK)]],
                pair_v.at[buf], gsem).wait()

        def drain_store():
            pltpu.make_async_copy(
                outt_v.at[0], out_hbm.at[0].at[:, pl.ds(0, BLK)], ssem).wait()

        fire_gather(0, 0)

        def pair_body(p, carry):
            for sub in range(2):
                k = 2 * p + sub
                buf = sub

                if sub == 0:
                    fire_gather(k + 1, 1 - buf)
                else:
                    @pl.when(p < n_pairs - 1)
                    def _():
                        fire_gather(k + 1, 1 - buf)

                drain_gather(buf)

                @pl.when(k >= 2)
                def _():
                    drain_store()

                blk_j = j0 + k * BLK
                t = blk_j // NB
                b0 = blk_j % NB
                koff = k * BLK
                iota = lax.iota(jnp.int32, 16)
                tok_g = [iota + g * 16 for g in range(BLK // 16)]
                hv_g = [hv_v[pl.ds(koff + g * 16, 16)]
                        for g in range(BLK // 16)]

                @plsc.parallel_loop(0, D // 4)
                def shuffle_d(d0):
                    for u in range(4):
                        lane_d = (d0 * 4 + u + iota) & 63
                        for g in range(BLK // 16):
                            vals = plsc.load_gather(
                                pair_v.at[buf], [tok_g[g], hv_g[g] + lane_d])
                            plsc.store_scatter(
                                outt_v.at[buf], [lane_d, tok_g[g]], vals)
                pltpu.async_copy(
                    outt_v.at[buf],
                    out_hbm.at[t].at[:, pl.ds(b0, BLK)], ssem)
            return carry

        lax.fori_loop(0, n_pairs, pair_body, 0)
        drain_store()
        drain_store()

    return gather_kernel


def kernel(seq, table):
    table2 = _build_relayout()(table.T)
    idx = seq.T.reshape(B)
    out = _build()(idx, table2)
    return out.transpose(2, 0, 1)


# speedup vs baseline: 1.8431x; 1.0003x over previous
import functools

import jax
import jax.numpy as jnp
from jax import lax
from jax.experimental import pallas as pl
from jax.experimental.pallas import tpu as pltpu
from jax.experimental.pallas import tpu_sc as plsc

D = 64
NT = 200
NB = 4096
B = NB * NT
BLK = 128


NCOL = 1000000 // 128 + 1


@functools.lru_cache(maxsize=None)
def _build_relayout():
    info = plsc.get_sparse_core_info()
    nc, ns = info.num_cores, info.num_subcores
    nw = nc * ns
    n_main = (NCOL - 5) // (2 * nw)

    mesh = plsc.VectorSubcoreMesh(core_axis_name="c", subcore_axis_name="s")

    @functools.partial(
        pl.kernel,
        out_type=jax.ShapeDtypeStruct((500000, 128), jnp.float32),
        mesh=mesh,
        compiler_params=pltpu.CompilerParams(
            use_tc_tiling_on_sc=True, needs_layout_passes=False,
            disable_bounds_checks=True),
        scratch_types=[
            pltpu.VMEM((2, D, 256), jnp.float32),
            pltpu.VMEM((2, 128, 128), jnp.float32),
            pltpu.VMEM((2, D * 256), jnp.float32),
            pltpu.VMEM((2, 128 * 128), jnp.float32),
            pltpu.SemaphoreType.DMA,
            pltpu.SemaphoreType.DMA,
        ],
    )
    def relayout_kernel(tt_hbm, tp_hbm, in_v, out_v, inf_v, outf_v,
                        isem, osem):
        wid = lax.axis_index("s") * nc + lax.axis_index("c")
        iota = lax.iota(jnp.int32, 16)
        hvec = lax.shift_right_logical(iota, 3)
        h64 = hvec * 64
        p8 = iota & 7

        def fire_in(ch, buf):
            pltpu.async_copy(
                tt_hbm.at[:, pl.ds(ch * 256, 256)], in_v.at[buf], isem)

        def drain_in(buf):
            pltpu.make_async_copy(
                tt_hbm.at[:, pl.ds(0, 256)], in_v.at[buf], isem).wait()

        def drain_out():
            pltpu.make_async_copy(
                out_v.at[0], tp_hbm.at[pl.ds(0, 128)], osem).wait()

        def shuffle(buf, npg):
            @plsc.parallel_loop(0, D // 4)
            def body(d0):
                for u in range(4):
                    dvec = (d0 * 4 + u + iota) & 63
                    dst_lane = h64 + dvec
                    for pg in range(npg):
                        pvec = p8 + pg * 8
                        src_lane = 2 * pvec + hvec
                        vals = plsc.load_gather(in_v.at[buf], [dvec, src_lane])
                        plsc.store_scatter(out_v.at[buf], [pvec, dst_lane], vals)

        fire_in(wid, 0)

        def pair_body(q, carry):
            for sub in range(2):
                ci = 2 * q + sub
                buf = sub
                ch = wid + ci * nw

                @pl.when(ci < n_main - 1)
                def _():
                    fire_in(ch + nw, 1 - buf)

                drain_in(buf)

                @pl.when(ci >= 2)
                def _():
                    drain_out()

                shuffle(buf, 16)
                pltpu.async_copy(
                    out_v.at[buf], tp_hbm.at[pl.ds(ch * 128, 128)], osem)
            return carry

        lax.fori_loop(0, n_main // 2, pair_body, 0)
        drain_out()
        drain_out()

        @pl.when(wid < 5)
        def _():
            c = (NCOL - 5) + wid
            pltpu.sync_copy(
                tt_hbm.at[:, pl.ds(c * 128, 128)],
                in_v.at[0].at[:, pl.ds(0, 128)])
            shuffle(0, 8)

            @pl.when(wid < 4)
            def _():
                pltpu.sync_copy(
                    out_v.at[0].at[pl.ds(0, D)],
                    tp_hbm.at[pl.ds(c * D, D)])

            @pl.when(wid == 4)
            def _():
                pltpu.sync_copy(
                    out_v.at[0].at[pl.ds(0, 32)],
                    tp_hbm.at[pl.ds(c * D, 32)])

    return relayout_kernel


@functools.lru_cache(maxsize=None)
def _build():
    info = plsc.get_sparse_core_info()
    nc, ns = info.num_cores, info.num_subcores
    nw = nc * ns
    b_per_w = B // nw
    n_blocks = b_per_w // BLK
    n_pairs = n_blocks // 2

    mesh = plsc.VectorSubcoreMesh(core_axis_name="c", subcore_axis_name="s")

    @functools.partial(
        pl.kernel,
        out_type=jax.ShapeDtypeStruct((NT, D, NB), jnp.float32),
        mesh=mesh,
        compiler_params=pltpu.CompilerParams(
            use_tc_tiling_on_sc=True, needs_layout_passes=False,
            disable_bounds_checks=True),
        scratch_types=[
            pltpu.VMEM((b_per_w,), jnp.int32),
            pltpu.VMEM((b_per_w,), jnp.int32),
            pltpu.VMEM((b_per_w,), jnp.int32),
            pltpu.VMEM((2, BLK, 128), jnp.float32),
            pltpu.VMEM((2, D, BLK), jnp.float32),
            pltpu.SemaphoreType.DMA,
            pltpu.SemaphoreType.DMA,
        ],
    )
    def gather_kernel(seq_hbm, table_hbm, out_hbm,
                      idx_v, idx2_v, hv_v, pair_v, outt_v, gsem, ssem):
        wid = lax.axis_index("s") * nc + lax.axis_index("c")
        j0 = wid * b_per_w
        pltpu.sync_copy(seq_hbm.at[pl.ds(j0, b_per_w)], idx_v)

        @plsc.parallel_loop(0, b_per_w // 16)
        def prep(i):
            x = idx_v[pl.ds(i * 16, 16)]
            idx2_v[pl.ds(i * 16, 16)] = lax.shift_right_logical(x, 1)
            hv_v[pl.ds(i * 16, 16)] = lax.shift_left(x & 1, 6)

        def fire_gather(k, buf):
            pltpu.async_copy(
                table_hbm.at[idx2_v.at[pl.ds(k * BLK, BLK)]],
                pair_v.at[buf], gsem)

        def drain_gather(buf):
            pltpu.make_async_copy(
                table_hbm.at[idx2_v.at[pl.ds(0, BLK)]],
                pair_v.at[buf], gsem).wait()

        def drain_store():
            pltpu.make_async_copy(
                outt_v.at[0], out_hbm.at[0].at[:, pl.ds(0, BLK)], ssem).wait()

        fire_gather(0, 0)

        def pair_body(p, carry):
            for sub in range(2):
                k = 2 * p + sub
                buf = sub

                if sub == 0:
                    fire_gather(k + 1, 1 - buf)
                else:
                    @pl.when(p < n_pairs - 1)
                    def _():
                        fire_gather(k + 1, 1 - buf)

                drain_gather(buf)

                @pl.when(k >= 2)
                def _():
                    drain_store()

                blk_j = j0 + k * BLK
                t = blk_j // NB
                b0 = blk_j % NB
                koff = k * BLK
                iota = lax.iota(jnp.int32, 16)
                tok_g = [iota + g * 16 for g in range(BLK // 16)]
                hv_g = [hv_v[pl.ds(koff + g * 16, 16)]
                        for g in range(BLK // 16)]

                @plsc.parallel_loop(0, D // 4)
                def shuffle_d(d0):
                    for u in range(4):
                        lane_d = (d0 * 4 + u + iota) & 63
                        for g in range(BLK // 16):
                            vals = plsc.load_gather(
                                pair_v.at[buf], [tok_g[g], hv_g[g] + lane_d])
                            plsc.store_scatter(
                                outt_v.at[buf], [lane_d, tok_g[g]], vals)
                pltpu.async_copy(
                    outt_v.at[buf],
                    out_hbm.at[t].at[:, pl.ds(b0, BLK)], ssem)
            return carry

        lax.fori_loop(0, n_pairs, pair_body, 0)
        drain_store()
        drain_store()

    return gather_kernel


def kernel(seq, table):
    table2 = _build_relayout()(table.T)
    idx = seq.T.reshape(B)
    out = _build()(idx, table2)
    return out.transpose(2, 0, 1)
